# Initial kernel scaffold; baseline (speedup 1.0000x reference)
import jax
import jax.numpy as jnp
from jax.experimental import pallas as pl


def kernel(x, edge_index, batch, W1, b1, g1, be1, W2, b2, g2, be2, l1W, l1b, l2W, l2b):
    raise NotImplementedError("write your pallas kernel here")


# speedup vs baseline: 6.6437x; 6.6437x over previous
import functools

import jax
import jax.numpy as jnp
from jax import lax
from jax.experimental import pallas as pl
from jax.experimental.pallas import tpu as pltpu
from jax.experimental.pallas import tpu_sc as plsc

N = 10000
E = 320000
D = 128
NB = 64

NC = 2
NS = 16
NW = NC * NS
EPW = E // NW
K = 80
CH = EPW // K
RPS = 624
TAIL = N - NS * RPS
ZR = 208

R = 2000
GRID = N // R

DW = 16


@functools.cache
def _make_sc_edge_agg():
    mesh = plsc.VectorSubcoreMesh(core_axis_name="c", subcore_axis_name="s")

    @functools.partial(
        pl.kernel, mesh=mesh,
        out_type=jax.ShapeDtypeStruct((NC, N, D), jnp.float32),
        scratch_types=[
            pltpu.VMEM((K,), jnp.int32),
            pltpu.VMEM((K,), jnp.int32),
            pltpu.VMEM((K, D), jnp.float32),
            pltpu.VMEM_SHARED((N, D), jnp.float32),
            pltpu.SemaphoreType.DMA,
        ],
    )
    def sc_edge_agg(hp_hbm, src_hbm, dst_hbm, zrows_hbm, out_hbm,
                    src_v, dst_v, rows_v, acc, sem):
        c = lax.axis_index("c")
        s = lax.axis_index("s")
        for j in range(RPS // ZR):
            pltpu.sync_copy(zrows_hbm, acc.at[pl.ds(s * RPS + j * ZR, ZR)])

        @pl.when(s == 0)
        def _():
            pltpu.sync_copy(zrows_hbm.at[pl.ds(0, TAIL)],
                            acc.at[pl.ds(NS * RPS, TAIL)])
        plsc.subcore_barrier()

        wid = c * NS + s

        def step(i, carry):
            base = wid * EPW + i * K
            pltpu.sync_copy(src_hbm.at[pl.ds(base, K)], src_v)
            pltpu.sync_copy(dst_hbm.at[pl.ds(base, K)], dst_v)
            pltpu.async_copy(hp_hbm.at[src_v], rows_v, sem).wait()
            pltpu.sync_copy(rows_v, acc.at[dst_v], add=True)
            return carry

        lax.fori_loop(0, CH, step, 0)
        plsc.subcore_barrier()
        pltpu.sync_copy(acc.at[pl.ds(s * RPS, RPS)],
                        out_hbm.at[c, pl.ds(s * RPS, RPS)])

        @pl.when(s == 0)
        def _():
            pltpu.sync_copy(acc.at[pl.ds(NS * RPS, TAIL)],
                            out_hbm.at[c, pl.ds(NS * RPS, TAIL)])

    return sc_edge_agg


def _sc_edge_agg(hp, src, dst, zrows):
    return _make_sc_edge_agg()(hp, src, dst, zrows)


@functools.cache
def _make_sc_deg():
    mesh = plsc.VectorSubcoreMesh(core_axis_name="c", subcore_axis_name="s")

    @functools.partial(
        pl.kernel, mesh=mesh,
        out_type=jax.ShapeDtypeStruct((NC, N, D), jnp.float32),
        scratch_types=[
            pltpu.VMEM((K,), jnp.int32),
            pltpu.VMEM((K, D), jnp.float32),
            pltpu.VMEM_SHARED((N, D), jnp.float32),
        ],
    )
    def sc_deg(dst_hbm, ones_hbm, zrows_hbm, out_hbm, dst_v, ones_v, acc):
        c = lax.axis_index("c")
        s = lax.axis_index("s")
        for j in range(RPS // ZR):
            pltpu.sync_copy(zrows_hbm, acc.at[pl.ds(s * RPS + j * ZR, ZR)])

        @pl.when(s == 0)
        def _():
            pltpu.sync_copy(zrows_hbm.at[pl.ds(0, TAIL)],
                            acc.at[pl.ds(NS * RPS, TAIL)])
        pltpu.sync_copy(ones_hbm, ones_v)
        plsc.subcore_barrier()

        wid = c * NS + s

        def step(i, carry):
            base = wid * EPW + i * K
            pltpu.sync_copy(dst_hbm.at[pl.ds(base, K)], dst_v)
            pltpu.sync_copy(ones_v, acc.at[dst_v], add=True)
            return carry

        lax.fori_loop(0, CH, step, 0)
        plsc.subcore_barrier()
        pltpu.sync_copy(acc.at[pl.ds(s * RPS, RPS)],
                        out_hbm.at[c, pl.ds(s * RPS, RPS)])

        @pl.when(s == 0)
        def _():
            pltpu.sync_copy(acc.at[pl.ds(NS * RPS, TAIL)],
                            out_hbm.at[c, pl.ds(NS * RPS, TAIL)])

    return sc_deg


def _sc_deg(dst, onesr, zrows):
    return _make_sc_deg()(dst, onesr, zrows)


def _lead_body(x_ref, w_ref, b_ref, degp_ref, h_ref, hp_ref, dis_ref):
    deg = degp_ref[0, :, 0:1] + degp_ref[1, :, 0:1] + 1.0
    dis = lax.rsqrt(deg)
    h = jnp.dot(x_ref[...], w_ref[...], preferred_element_type=jnp.float32)
    h = h + b_ref[...]
    h_ref[...] = h
    hp_ref[...] = h * dis
    dis_ref[...] = dis


def _tc_lead(x, w, b, degp):
    return pl.pallas_call(
        _lead_body,
        grid=(GRID,),
        in_specs=[
            pl.BlockSpec((R, D), lambda i: (i, 0)),
            pl.BlockSpec((D, D), lambda i: (0, 0)),
            pl.BlockSpec((1, D), lambda i: (0, 0)),
            pl.BlockSpec((NC, R, D), lambda i: (0, i, 0)),
        ],
        out_specs=[
            pl.BlockSpec((R, D), lambda i: (i, 0)),
            pl.BlockSpec((R, D), lambda i: (i, 0)),
            pl.BlockSpec((R, 1), lambda i: (i, 0)),
        ],
        out_shape=[
            jax.ShapeDtypeStruct((N, D), jnp.float32),
            jax.ShapeDtypeStruct((N, D), jnp.float32),
            jax.ShapeDtypeStruct((N, 1), jnp.float32),
        ],
    )(x, w, b, degp)


def _relu_stats_body(p_ref, h_ref, dis_ref, y_ref, ssum_ref, ssq_ref):
    dis = dis_ref[...]
    agg = dis * (p_ref[0] + p_ref[1]) + (dis * dis) * h_ref[...]
    y = jnp.maximum(agg, 0.0)
    y_ref[...] = y

    @pl.when(pl.program_id(0) == 0)
    def _():
        ssum_ref[...] = jnp.zeros_like(ssum_ref)
        ssq_ref[...] = jnp.zeros_like(ssq_ref)

    ssum_ref[...] += jnp.sum(y, axis=0, keepdims=True)
    ssq_ref[...] += jnp.sum(y * y, axis=0, keepdims=True)


def _tc_relu_stats(p, h, dis):
    return pl.pallas_call(
        _relu_stats_body,
        grid=(GRID,),
        in_specs=[
            pl.BlockSpec((NC, R, D), lambda i: (0, i, 0)),
            pl.BlockSpec((R, D), lambda i: (i, 0)),
            pl.BlockSpec((R, 1), lambda i: (i, 0)),
        ],
        out_specs=[
            pl.BlockSpec((R, D), lambda i: (i, 0)),
            pl.BlockSpec((1, D), lambda i: (0, 0)),
            pl.BlockSpec((1, D), lambda i: (0, 0)),
        ],
        out_shape=[
            jax.ShapeDtypeStruct((N, D), jnp.float32),
            jax.ShapeDtypeStruct((1, D), jnp.float32),
            jax.ShapeDtypeStruct((1, D), jnp.float32),
        ],
    )(p, h, dis)


def _bn_matmul_body(y_ref, ssum_ref, ssq_ref, g_ref, be_ref, w_ref, b_ref,
                    dis_ref, h_ref, hp_ref):
    mu = ssum_ref[...] / N
    var = ssq_ref[...] / N - mu * mu
    rstd = lax.rsqrt(var + 1e-5)
    xn = (y_ref[...] - mu) * (rstd * g_ref[...]) + be_ref[...]
    h = jnp.dot(xn, w_ref[...], preferred_element_type=jnp.float32)
    h = h + b_ref[...]
    h_ref[...] = h
    hp_ref[...] = h * dis_ref[...]


def _tc_bn_matmul(y, ssum, ssq, g, be, w, b, dis):
    return pl.pallas_call(
        _bn_matmul_body,
        grid=(GRID,),
        in_specs=[
            pl.BlockSpec((R, D), lambda i: (i, 0)),
            pl.BlockSpec((1, D), lambda i: (0, 0)),
            pl.BlockSpec((1, D), lambda i: (0, 0)),
            pl.BlockSpec((1, D), lambda i: (0, 0)),
            pl.BlockSpec((1, D), lambda i: (0, 0)),
            pl.BlockSpec((D, D), lambda i: (0, 0)),
            pl.BlockSpec((1, D), lambda i: (0, 0)),
            pl.BlockSpec((R, 1), lambda i: (i, 0)),
        ],
        out_specs=[
            pl.BlockSpec((R, D), lambda i: (i, 0)),
            pl.BlockSpec((R, D), lambda i: (i, 0)),
        ],
        out_shape=[
            jax.ShapeDtypeStruct((N, D), jnp.float32),
            jax.ShapeDtypeStruct((N, D), jnp.float32),
        ],
    )(y, ssum, ssq, g, be, w, b, dis)


def _bn_pool_body(y_ref, ssum_ref, ssq_ref, g_ref, be_ref, batch_ref,
                  segs_ref, cnt_ref):
    mu = ssum_ref[...] / N
    var = ssq_ref[...] / N - mu * mu
    rstd = lax.rsqrt(var + 1e-5)
    xn = (y_ref[...] - mu) * (rstd * g_ref[...]) + be_ref[...]
    seg_ids = lax.broadcasted_iota(jnp.int32, (1, NB), 1)
    oneh = jnp.where(batch_ref[...] == seg_ids, 1.0, 0.0)

    @pl.when(pl.program_id(0) == 0)
    def _():
        segs_ref[...] = jnp.zeros_like(segs_ref)
        cnt_ref[...] = jnp.zeros_like(cnt_ref)

    segs_ref[...] += lax.dot_general(oneh, xn, (((0,), (0,)), ((), ())),
                                     preferred_element_type=jnp.float32)
    ones_col = jnp.ones((R, 1), jnp.float32)
    cnt_ref[...] += lax.dot_general(oneh, ones_col, (((0,), (0,)), ((), ())),
                                    preferred_element_type=jnp.float32)


def _tc_bn_pool(y, ssum, ssq, g, be, batch2):
    return pl.pallas_call(
        _bn_pool_body,
        grid=(GRID,),
        in_specs=[
            pl.BlockSpec((R, D), lambda i: (i, 0)),
            pl.BlockSpec((1, D), lambda i: (0, 0)),
            pl.BlockSpec((1, D), lambda i: (0, 0)),
            pl.BlockSpec((1, D), lambda i: (0, 0)),
            pl.BlockSpec((1, D), lambda i: (0, 0)),
            pl.BlockSpec((R, 1), lambda i: (i, 0)),
        ],
        out_specs=[
            pl.BlockSpec((NB, D), lambda i: (0, 0)),
            pl.BlockSpec((NB, 1), lambda i: (0, 0)),
        ],
        out_shape=[
            jax.ShapeDtypeStruct((NB, D), jnp.float32),
            jax.ShapeDtypeStruct((NB, 1), jnp.float32),
        ],
    )(y, ssum, ssq, g, be, batch2)


def _head_body(segs_ref, cnt_ref, w1_ref, b1_ref, w2_ref, b2_ref, out_ref):
    pooled = segs_ref[...] / jnp.maximum(cnt_ref[...], 1.0)
    o = jnp.dot(pooled, w1_ref[...], preferred_element_type=jnp.float32)
    o = o + b1_ref[...]
    o = jnp.dot(o, w2_ref[...], preferred_element_type=jnp.float32)
    o = o + b2_ref[...]
    m = jnp.max(o, axis=-1, keepdims=True)
    lse = m + jnp.log(jnp.sum(jnp.exp(o - m), axis=-1, keepdims=True))
    out_ref[...] = o - lse


def _tc_head(segs, cnt, l1W, l1b, l2W, l2b):
    return pl.pallas_call(
        _head_body,
        out_shape=jax.ShapeDtypeStruct((NB, 4), jnp.float32),
    )(segs, cnt, l1W, l1b.reshape(1, -1), l2W, l2b.reshape(1, -1))


def kernel(x, edge_index, batch, W1, b1, g1, be1, W2, b2, g2, be2,
           l1W, l1b, l2W, l2b):
    src = edge_index[0]
    dst = edge_index[1]
    batch2 = batch.astype(jnp.int32).reshape(N, 1)
    zrows = jnp.zeros((ZR, D), jnp.float32)
    onesr = jnp.ones((K, D), jnp.float32)

    degp = _sc_deg(dst, onesr, zrows)

    Ws = [W1[0], W1[1], W1[2], W2[0], W2[1], W2[2]]
    bs = [b1[0].reshape(1, D), b1[1].reshape(1, D), b1[2].reshape(1, D),
          b2[0].reshape(1, D), b2[1].reshape(1, D), b2[2].reshape(1, D)]
    gs = [g1[0].reshape(1, D), g1[1].reshape(1, D), g1[2].reshape(1, D),
          g2[0].reshape(1, D), g2[1].reshape(1, D), g2[2].reshape(1, D)]
    bes = [be1[0].reshape(1, D), be1[1].reshape(1, D), be1[2].reshape(1, D),
           be2[0].reshape(1, D), be2[1].reshape(1, D), be2[2].reshape(1, D)]

    h, hp, dis = _tc_lead(x, Ws[0], bs[0], degp)
    segs = cnt = None
    for l in range(6):
        p = _sc_edge_agg(hp, src, dst, zrows)
        y, ssum, ssq = _tc_relu_stats(p, h, dis)
        if l < 5:
            h, hp = _tc_bn_matmul(y, ssum, ssq, gs[l], bes[l],
                                  Ws[l + 1], bs[l + 1], dis)
        else:
            segs, cnt = _tc_bn_pool(y, ssum, ssq, gs[l], bes[l], batch2)
    return _tc_head(segs, cnt, l1W, l1b, l2W, l2b)
